# speedup vs baseline: 2.6963x; 2.6963x over previous
import functools

import jax
import jax.numpy as jnp
from jax import lax
from jax.experimental import pallas as pl
from jax.experimental.pallas import tpu as pltpu
from jax.experimental.pallas import tpu_sc as plsc

_NC = 2
_NS = 16


def _mm_body(a_ref, b_ref, o_ref):
    o_ref[...] = lax.dot_general(
        a_ref[...], b_ref[...],
        dimension_numbers=(((1,), (1,)), ((), ())),
        preferred_element_type=jnp.float32,
    )


def _project_table(A, B, block_rows):
    vocab, k = A.shape
    d = B.shape[0]
    grid = vocab // block_rows
    return pl.pallas_call(
        _mm_body,
        grid=(grid,),
        in_specs=[
            pl.BlockSpec((block_rows, k), lambda i: (i, 0)),
            pl.BlockSpec((d, k), lambda i: (0, 0)),
        ],
        out_specs=pl.BlockSpec((block_rows, d), lambda i: (i, 0)),
        out_shape=jax.ShapeDtypeStruct((vocab, d), jnp.float32),
    )(A, B)


def _make_gather(n_idx, d, per_w, chunk):
    n_chunks = per_w // chunk
    mesh = plsc.VectorSubcoreMesh(core_axis_name="c", subcore_axis_name="s")

    @functools.partial(
        pl.kernel,
        out_type=jax.ShapeDtypeStruct((n_idx, d), jnp.float32),
        mesh=mesh,
        scratch_types=[
            pltpu.VMEM((per_w,), jnp.int32),
            pltpu.VMEM((chunk, d), jnp.float32),
            pltpu.VMEM((chunk, d), jnp.float32),
            pltpu.SemaphoreType.DMA,
            pltpu.SemaphoreType.DMA,
        ],
    )
    def gather(table_hbm, idx_hbm, out_hbm, idx_v, rows0, rows1, sem0, sem1):
        wid = lax.axis_index("s") * _NC + lax.axis_index("c")
        base = wid * per_w
        pltpu.sync_copy(idx_hbm.at[pl.ds(base, per_w)], idx_v)

        pltpu.async_copy(table_hbm.at[idx_v.at[pl.ds(0, chunk)]], rows0, sem0)

        def step(i, _):
            slot = lax.rem(i, 2)

            def run(cur_rows, cur_sem, nxt_rows, nxt_sem):
                @pl.when(i + 1 < n_chunks)
                def _():
                    pltpu.async_copy(
                        table_hbm.at[idx_v.at[pl.ds((i + 1) * chunk, chunk)]],
                        nxt_rows, nxt_sem,
                    )
                pltpu.make_async_copy(
                    table_hbm.at[idx_v.at[pl.ds(i * chunk, chunk)]],
                    cur_rows, cur_sem,
                ).wait()
                pltpu.sync_copy(
                    cur_rows, out_hbm.at[pl.ds(base + i * chunk, chunk)]
                )

            @pl.when(slot == 0)
            def _():
                run(rows0, sem0, rows1, sem1)

            @pl.when(slot == 1)
            def _():
                run(rows1, sem1, rows0, sem0)

            return 0

        lax.fori_loop(0, n_chunks, step, 0)

    return gather


def kernel(token_ids, A, B):
    batch, hist = token_ids.shape
    vocab, k = A.shape
    d = B.shape[0]

    table = _project_table(A, B, block_rows=1000)

    n_idx = batch * hist
    flat_ids = token_ids.reshape(n_idx).astype(jnp.int32)

    per_w = n_idx // (_NC * _NS)
    chunk = 320
    gather = _make_gather(n_idx, d, per_w, chunk)
    out = gather(table, flat_ids)
    return out.reshape(batch, hist, d)
